# baseline (speedup 1.0000x reference)
import jax
import jax.numpy as jnp
from jax import lax
from jax.experimental import pallas as pl
from jax.experimental.pallas import tpu as pltpu, tpu_sc as plsc

VOCAB = 1000000
EMBED_DIM = 64
BATCH = 16384
FIELDS = 26

NC = 2
NS = 16
NW = NC * NS

CHUNK = 128
NCHUNK = BATCH // CHUNK
TASKS = FIELDS * NCHUNK
TASKS_PER_W = TASKS // NW
STEPS = TASKS_PER_W // 2


def _embed_kernel(idx_hbm, table_hbm, out_hbm,
                  ib, q0, q1, g0, g1, t0b, t1b, gsems, wsems):
    qs = [q0, q1]
    gs = [g0, g1]
    ts = [t0b, t1b]
    wid = lax.axis_index("s") * NC + lax.axis_index("c")
    t0 = wid * TASKS_PER_W

    def task_fc(k):
        t = t0 + k
        return t // NCHUNK, t % NCHUNK

    def stage_and_fire(k, b):
        f, c = task_fc(k)
        pltpu.sync_copy(idx_hbm.at[f, pl.ds(c * CHUNK, CHUNK)], ib.at[b])
        for j in range(CHUNK // 16):
            qs[b][pl.ds(j * 16, 16)] = ib[b, pl.ds(j * 16, 16)] >> 1
        pltpu.async_copy(table_hbm.at[qs[b]], gs[b], gsems.at[b])

    for b in range(2):
        stage_and_fire(b, b)

    iota = lax.broadcasted_iota(jnp.int32, (16,), 0)
    a_idx = [(d0 + iota) >> 3 for d0 in range(0, EMBED_DIM, 16)]
    s_idx = [(d0 + iota) & 7 for d0 in range(0, EMBED_DIM, 16)]

    def transpose_block(b):
        def grp(lg, carry):
            pv = (ib[b, pl.ds(lg * 16, 16)] & 1) * EMBED_DIM
            base = lg * 16
            for j in range(16):
                col = jnp.full((16,), 1, jnp.int32) * (base + j)
                half = pv[j]
                for q in range(EMBED_DIM // 16):
                    x = gs[b][base + j, pl.ds(half + q * 16, 16)]
                    plsc.store_scatter(ts[b], [a_idx[q], s_idx[q], col], x)
            return carry
        lax.fori_loop(0, CHUNK // 16, grp, 0)

    def wb_dst(f, c):
        return out_hbm.at[f, :, c]

    def step(i, carry):
        for b in range(2):
            k = i * 2 + b
            f, c = task_fc(k)
            @pl.when(i > 0)
            def _():
                pltpu.make_async_copy(
                    ts[b].at[:, :, pl.ds(0, CHUNK)], wb_dst(f, c),
                    wsems.at[b]).wait()
            pltpu.make_async_copy(
                table_hbm.at[qs[b]], gs[b], gsems.at[b]).wait()
            transpose_block(b)
            pltpu.async_copy(ts[b].at[:, :, pl.ds(0, CHUNK)], wb_dst(f, c),
                             wsems.at[b])
            @pl.when(i < STEPS - 1)
            def _():
                stage_and_fire(k + 2, b)
        return carry

    lax.fori_loop(0, STEPS, step, 0)
    for b in range(2):
        k = (STEPS - 1) * 2 + b
        f, c = task_fc(k)
        pltpu.make_async_copy(
            ts[b].at[:, :, pl.ds(0, CHUNK)], wb_dst(f, c), wsems.at[b]).wait()


def kernel(embed_input, weight):
    idx_t = embed_input.T
    w2 = weight.reshape(VOCAB // 2, 2 * EMBED_DIM)
    mesh = plsc.VectorSubcoreMesh(core_axis_name="c", subcore_axis_name="s")
    o5 = pl.kernel(
        _embed_kernel,
        out_type=jax.ShapeDtypeStruct((FIELDS, 8, NCHUNK, 8, CHUNK),
                                      jnp.float32),
        mesh=mesh,
        compiler_params=pltpu.CompilerParams(use_tc_tiling_on_sc=True,
                                             needs_layout_passes=False),
        scratch_types=[
            pltpu.VMEM((2, CHUNK), jnp.int32),
            pltpu.VMEM((CHUNK,), jnp.int32),
            pltpu.VMEM((CHUNK,), jnp.int32),
            pltpu.VMEM((CHUNK, 2 * EMBED_DIM), jnp.float32),
            pltpu.VMEM((CHUNK, 2 * EMBED_DIM), jnp.float32),
            pltpu.VMEM((8, 8, CHUNK + 1), jnp.float32),
            pltpu.VMEM((8, 8, CHUNK + 1), jnp.float32),
            pltpu.SemaphoreType.DMA((2,)),
            pltpu.SemaphoreType.DMA((2,)),
        ],
    )(idx_t, w2)
    return o5.transpose(2, 4, 0, 1, 3).reshape(BATCH, FIELDS, EMBED_DIM)
